# Initial kernel scaffold; baseline (speedup 1.0000x reference)
import jax
import jax.numpy as jnp
from jax.experimental import pallas as pl


def kernel(x, W1, b1, W2, b2, W_disc, edge_index, perm, cluster_info, cluster_num):
    raise NotImplementedError("write your pallas kernel here")


# speedup vs baseline: 3.5540x; 3.5540x over previous
import functools

import jax
import jax.numpy as jnp
from jax import lax
from jax.experimental import pallas as pl
from jax.experimental.pallas import tpu as pltpu
from jax.experimental.pallas import tpu_sc as plsc

_NC = 2
_NS = 16
_NW = _NC * _NS
_CHUNK = 128
_WV = 128


def _proj_body(x_ref, w_ref, o_ref):
    z = jnp.dot(x_ref[...], w_ref[...], preferred_element_type=jnp.float32)
    pad = jnp.zeros((z.shape[0], _WV - z.shape[1]), jnp.float32)
    o_ref[...] = jnp.concatenate([z, pad], axis=1)


def _pack_body(zw_ref, zp_ref, o_ref):
    h = zw_ref.shape[1] // 4
    z = zw_ref[:, 0:h]
    zn = zp_ref[:, 0:h]
    pad = jnp.zeros((z.shape[0], _WV - 2 * h), jnp.float32)
    o_ref[...] = jnp.concatenate([z, zn, pad], axis=1)


def _mid_body(z1_ref, acc_ref, dc_ref, b1_ref, w2_ref, y_ref, rdeg_ref):
    h = w2_ref.shape[0]
    deg = jnp.maximum(dc_ref[0, :, 0:1] + dc_ref[1, :, 0:1], 1.0)
    r = 1.0 / deg
    rdeg_ref[...] = r
    aggp = (acc_ref[0, :, 0:h] + acc_ref[1, :, 0:h]) * r
    aggn = (acc_ref[0, :, h:2 * h] + acc_ref[1, :, h:2 * h]) * r
    h1p = jnp.maximum(z1_ref[:, 0:h] + aggp + b1_ref[...], 0.0)
    h1n = jnp.maximum(z1_ref[:, h:2 * h] + aggn + b1_ref[...], 0.0)
    yp = jnp.dot(h1p, w2_ref[...], preferred_element_type=jnp.float32)
    yn = jnp.dot(h1n, w2_ref[...], preferred_element_type=jnp.float32)
    pad = jnp.zeros((yp.shape[0], _WV - 2 * h), jnp.float32)
    y_ref[...] = jnp.concatenate([yp, yn, pad], axis=1)


def _softplus(v):
    return jnp.maximum(v, 0.0) + jnp.log(1.0 + jnp.exp(-jnp.abs(v)))


def _final_body(inv_denom, y_ref, acc_ref, rdeg_ref, dc_ref, b2_ref, wd_ref,
                o_ref):
    h = wd_ref.shape[0]
    r = rdeg_ref[...]
    aggp = (acc_ref[0, :, 0:h] + acc_ref[1, :, 0:h]) * r
    aggn = (acc_ref[0, :, h:2 * h] + acc_ref[1, :, h:2 * h]) * r
    p2 = jnp.maximum(y_ref[:, 0:h] + aggp + b2_ref[...], 0.0)
    n2 = jnp.maximum(y_ref[:, h:2 * h] + aggn + b2_ref[...], 0.0)
    summary = jax.nn.sigmoid(jnp.mean(p2, axis=0, keepdims=True))
    ws = jnp.sum(wd_ref[...] * summary, axis=1, keepdims=True)
    pos = jnp.dot(p2, ws, preferred_element_type=jnp.float32)
    neg = jnp.dot(n2, ws, preferred_element_type=jnp.float32)
    cnt = dc_ref[0, :, 64:65] + dc_ref[1, :, 64:65]
    tot = jnp.sum(cnt * (_softplus(-pos) + _softplus(neg)), keepdims=True)
    o_ref[...] = tot.reshape(1, 1) * inv_denom


def kernel(x, W1, b1, W2, b2, W_disc, edge_index, perm, cluster_info, cluster_num):
    f32, i32 = jnp.float32, jnp.int32
    N, D = x.shape
    H = W1.shape[0]
    E = edge_index.shape[1]
    C, P = cluster_info.shape

    rw = -(-N // (_NW * _CHUNK)) * _CHUNK
    ipad = _NW * rw
    npad = -(-(N + 8) // (_NS * 8)) * (_NS * 8)
    slc = npad // _NS
    ew = -(-E // (_NW * _CHUNK)) * _CHUNK
    epad = _NW * ew

    src_pad = jnp.concatenate([edge_index[0].astype(i32),
                               jnp.zeros((epad - E,), i32)])
    dst_pad = jnp.concatenate([edge_index[1].astype(i32),
                               jnp.full((epad - E,), N, i32)])
    perm_pad = jnp.concatenate([perm.astype(i32), jnp.zeros((ipad - N,), i32)])
    ci_pad = jnp.concatenate([cluster_info.reshape(-1).astype(i32),
                              jnp.full((ipad - C * P,), N, i32)])
    lane = jnp.arange(_WV)
    deg_ones = jnp.where(lane < 64, 1.0, 0.0).astype(f32) * jnp.ones((_CHUNK, 1), f32)
    cnt_ones = jnp.where(lane >= 64, 1.0, 0.0).astype(f32) * jnp.ones((_CHUNK, 1), f32)
    zeros_h = jnp.zeros((slc, _WV), f32)
    w1t = W1.T
    w2t = W2.T
    b1r = b1.reshape(1, H)
    b2r = b2.reshape(1, H)

    mesh = plsc.VectorSubcoreMesh(core_axis_name="c", subcore_axis_name="s",
                                  num_cores=_NC, num_subcores=_NS)

    def prep_body(zw_ref, perm_ref, dstp_ref, ci_ref, dones_ref, cones_ref,
                  zeros_ref, zperm_out, dc_out,
                  dc_sh, idx_v, rows_v, dones_v, cones_v, sem):
        c = lax.axis_index("c")
        s = lax.axis_index("s")
        wid = s * _NC + c
        pltpu.sync_copy(zeros_ref, dc_sh.at[pl.ds(s * slc, slc)])
        pltpu.sync_copy(dones_ref, dones_v)
        pltpu.sync_copy(cones_ref, cones_v)
        plsc.subcore_barrier()
        for j in range(rw // _CHUNK):
            b = wid * rw + j * _CHUNK
            pltpu.sync_copy(perm_ref.at[pl.ds(b, _CHUNK)], idx_v)
            pltpu.async_copy(zw_ref.at[idx_v], rows_v, sem).wait()
            pltpu.sync_copy(rows_v, zperm_out.at[pl.ds(b, _CHUNK)])

        def deg_step(j, carry):
            b = wid * ew + j * _CHUNK
            pltpu.sync_copy(dstp_ref.at[pl.ds(b, _CHUNK)], idx_v)
            pltpu.sync_copy(dones_v, dc_sh.at[idx_v], add=True)
            return carry

        lax.fori_loop(0, ew // _CHUNK, deg_step, 0)
        for j in range(rw // _CHUNK):
            b = wid * rw + j * _CHUNK
            pltpu.sync_copy(ci_ref.at[pl.ds(b, _CHUNK)], idx_v)
            pltpu.sync_copy(cones_v, dc_sh.at[idx_v], add=True)
        plsc.subcore_barrier()
        pltpu.sync_copy(dc_sh.at[pl.ds(s * slc, slc)],
                        dc_out.at[c, pl.ds(s * slc, slc)])

    prep = pl.kernel(
        prep_body,
        out_type=[
            jax.ShapeDtypeStruct((ipad, _WV), f32),
            jax.ShapeDtypeStruct((_NC, npad, _WV), f32),
        ],
        mesh=mesh,
        scratch_types=[
            pltpu.VMEM_SHARED((npad, _WV), f32),
            pltpu.VMEM((_CHUNK,), i32),
            pltpu.VMEM((_CHUNK, _WV), f32),
            pltpu.VMEM((_CHUNK, _WV), f32),
            pltpu.VMEM((_CHUNK, _WV), f32),
            pltpu.SemaphoreType.DMA,
        ],
    )

    def edge_body(vals_ref, srcp_ref, dstp_ref, zeros_ref, acc_out,
                  acc_sh, idxs_v, idxd_v, rows_v, sem):
        c = lax.axis_index("c")
        s = lax.axis_index("s")
        wid = s * _NC + c
        pltpu.sync_copy(zeros_ref, acc_sh.at[pl.ds(s * slc, slc)])
        plsc.subcore_barrier()

        def step(j, carry):
            b = wid * ew + j * _CHUNK
            pltpu.sync_copy(srcp_ref.at[pl.ds(b, _CHUNK)], idxs_v)
            pltpu.sync_copy(dstp_ref.at[pl.ds(b, _CHUNK)], idxd_v)
            pltpu.async_copy(vals_ref.at[idxs_v], rows_v, sem).wait()
            pltpu.sync_copy(rows_v, acc_sh.at[idxd_v], add=True)
            return carry

        lax.fori_loop(0, ew // _CHUNK, step, 0)
        plsc.subcore_barrier()
        pltpu.sync_copy(acc_sh.at[pl.ds(s * slc, slc)],
                        acc_out.at[c, pl.ds(s * slc, slc)])

    edge_pass = pl.kernel(
        edge_body,
        out_type=jax.ShapeDtypeStruct((_NC, npad, _WV), f32),
        mesh=mesh,
        scratch_types=[
            pltpu.VMEM_SHARED((npad, _WV), f32),
            pltpu.VMEM((_CHUNK,), i32),
            pltpu.VMEM((_CHUNK,), i32),
            pltpu.VMEM((_CHUNK, _WV), f32),
            pltpu.SemaphoreType.DMA,
        ],
    )

    zw = pl.pallas_call(
        _proj_body,
        out_shape=jax.ShapeDtypeStruct((N, _WV), f32),
    )(x, w1t)

    zperm, dcp = prep(zw, perm_pad, dst_pad, ci_pad, deg_ones, cnt_ones, zeros_h)

    z1 = pl.pallas_call(
        _pack_body,
        out_shape=jax.ShapeDtypeStruct((N, _WV), f32),
    )(zw, zperm[:N])

    acc1 = edge_pass(z1, src_pad, dst_pad, zeros_h)

    y1, rdeg = pl.pallas_call(
        _mid_body,
        out_shape=[
            jax.ShapeDtypeStruct((N, _WV), f32),
            jax.ShapeDtypeStruct((N, 1), f32),
        ],
    )(z1, acc1[:, :N], dcp[:, :N], b1r, w2t)

    acc2 = edge_pass(y1, src_pad, dst_pad, zeros_h)

    out = pl.pallas_call(
        functools.partial(_final_body, 1.0 / float(C * P)),
        out_shape=jax.ShapeDtypeStruct((1, 1), f32),
    )(y1, acc2[:, :N], rdeg, dcp[:, :N], b2r, W_disc)
    return out[0, 0]
